# speedup vs baseline: 1.1666x; 1.1666x over previous
import functools
import math

import jax
import jax.numpy as jnp
from jax import lax
from jax.experimental import pallas as pl
from jax.experimental.pallas import tpu as pltpu
from jax.experimental.pallas import tpu_sc as plsc

D_MODEL = 512
SCALE = float(math.sqrt(D_MODEL))

NUM_CORES = 2
NUM_SUBCORES = 16
NW = NUM_CORES * NUM_SUBCORES

B_TOTAL = 1024 * 200
BPW = B_TOTAL // NW
C = 64
NCHUNK = BPW // C

_mesh = plsc.VectorSubcoreMesh(core_axis_name="c", subcore_axis_name="s")


@functools.partial(
    pl.kernel,
    mesh=_mesh,
    out_type=jax.ShapeDtypeStruct((B_TOTAL, D_MODEL), jnp.float32),
    scratch_types=[
        pltpu.VMEM((BPW,), jnp.int32),
        pltpu.VMEM((C, D_MODEL), jnp.float32),
        pltpu.SemaphoreType.DMA,
    ],
)
def _emb_lookup(idx_hbm, table_hbm, out_hbm, idx_v, rows_v, gsem):
    wid = lax.axis_index("s") * NUM_CORES + lax.axis_index("c")
    base = wid * BPW
    pltpu.sync_copy(idx_hbm.at[pl.ds(base, BPW)], idx_v)

    def chunk_body(g, carry):
        off = g * C
        pltpu.async_copy(
            table_hbm.at[idx_v.at[pl.ds(off, C)]], rows_v, gsem
        ).wait()

        def row_body(r, rc):
            for j in range(D_MODEL // 16):
                sl = pl.ds(j * 16, 16)
                rows_v[r, sl] = rows_v[r, sl] * SCALE
            return rc

        lax.fori_loop(0, C, row_body, 0, unroll=False)

        pltpu.sync_copy(rows_v, out_hbm.at[pl.ds(base + off, C)])
        return carry

    lax.fori_loop(0, NCHUNK, chunk_body, 0, unroll=False)


def kernel(x, table):
    flat_idx = x.reshape(-1)
    out = _emb_lookup(flat_idx, table)
    return out.reshape(x.shape + (D_MODEL,))


# speedup vs baseline: 1.8034x; 1.5459x over previous
import functools
import math

import jax
import jax.numpy as jnp
from jax import lax
from jax.experimental import pallas as pl
from jax.experimental.pallas import tpu as pltpu
from jax.experimental.pallas import tpu_sc as plsc

D_MODEL = 512
SCALE = float(math.sqrt(D_MODEL))

NUM_CORES = 2
NUM_SUBCORES = 16
NW = NUM_CORES * NUM_SUBCORES

B_TOTAL = 1024 * 200
BPW = B_TOTAL // NW
C = 40
NCHUNK = BPW // C
T = NCHUNK // 2

_mesh = plsc.VectorSubcoreMesh(core_axis_name="c", subcore_axis_name="s")


@functools.partial(
    pl.kernel,
    mesh=_mesh,
    out_type=jax.ShapeDtypeStruct((B_TOTAL, D_MODEL), jnp.float32),
    scratch_types=[
        pltpu.VMEM((BPW,), jnp.int32),
        pltpu.VMEM((C, D_MODEL), jnp.float32),
        pltpu.VMEM((C, D_MODEL), jnp.float32),
        pltpu.VMEM((C, D_MODEL), jnp.float32),
        pltpu.VMEM((C, D_MODEL), jnp.float32),
        pltpu.SemaphoreType.DMA,
        pltpu.SemaphoreType.DMA,
        pltpu.SemaphoreType.DMA,
        pltpu.SemaphoreType.DMA,
    ],
)
def _emb_lookup(idx_hbm, table_hbm, out_hbm, idx_v,
                in0, in1, out0, out1, gs0, gs1, ss0, ss1):
    wid = lax.axis_index("s") * NUM_CORES + lax.axis_index("c")
    base = wid * BPW
    pltpu.sync_copy(idx_hbm.at[pl.ds(base, BPW)], idx_v)

    ins, outs = (in0, in1), (out0, out1)
    gss, sss = (gs0, gs1), (ss0, ss1)

    def start_gather(g, b):
        pltpu.async_copy(table_hbm.at[idx_v.at[pl.ds(g * C, C)]], ins[b], gss[b])

    def wait_gather(b):
        pltpu.make_async_copy(
            table_hbm.at[idx_v.at[pl.ds(0, C)]], ins[b], gss[b]
        ).wait()

    def start_scatter(g, b):
        pltpu.async_copy(outs[b], out_hbm.at[pl.ds(base + g * C, C)], sss[b])

    def wait_scatter(b):
        pltpu.make_async_copy(outs[b], out_hbm.at[pl.ds(base, C)], sss[b]).wait()

    def scale(b):
        def row_body(r, rc):
            for j in range(D_MODEL // 16):
                sl = pl.ds(j * 16, 16)
                outs[b][r, sl] = ins[b][r, sl] * SCALE
            return rc

        lax.fori_loop(0, C, row_body, 0, unroll=False)

    start_gather(0, 0)
    start_gather(1, 1)
    for b in (0, 1):
        wait_gather(b)
        scale(b)
        start_scatter(b, b)
        start_gather(b + 2, b)

    def steady(t, carry):
        for b in (0, 1):
            g = 2 * t + b
            wait_gather(b)
            wait_scatter(b)
            scale(b)
            start_scatter(g, b)
            start_gather(g + 2, b)
        return carry

    lax.fori_loop(1, T - 1, steady, 0, unroll=False)

    for b in (0, 1):
        g = NCHUNK - 2 + b
        wait_gather(b)
        wait_scatter(b)
        scale(b)
        start_scatter(g, b)
    wait_scatter(0)
    wait_scatter(1)


def kernel(x, table):
    flat_idx = x.reshape(-1)
    out = _emb_lookup(flat_idx, table)
    return out.reshape(x.shape + (D_MODEL,))


# speedup vs baseline: 1.8061x; 1.0015x over previous
import functools
import math

import jax
import jax.numpy as jnp
from jax import lax
from jax.experimental import pallas as pl
from jax.experimental.pallas import tpu as pltpu
from jax.experimental.pallas import tpu_sc as plsc

D_MODEL = 512
SCALE = float(math.sqrt(D_MODEL))

NUM_CORES = 2
NUM_SUBCORES = 16
NW = NUM_CORES * NUM_SUBCORES

B_TOTAL = 1024 * 200
BPW = B_TOTAL // NW
C = 40
NCHUNK = BPW // C
NBUF = 4
T = NCHUNK // NBUF

_mesh = plsc.VectorSubcoreMesh(core_axis_name="c", subcore_axis_name="s")


@functools.partial(
    pl.kernel,
    mesh=_mesh,
    out_type=jax.ShapeDtypeStruct((B_TOTAL, D_MODEL), jnp.float32),
    scratch_types=[
        pltpu.VMEM((BPW,), jnp.int32),
        pltpu.VMEM((C, D_MODEL), jnp.float32),
        pltpu.VMEM((C, D_MODEL), jnp.float32),
        pltpu.VMEM((C, D_MODEL), jnp.float32),
        pltpu.VMEM((C, D_MODEL), jnp.float32),
        pltpu.SemaphoreType.DMA,
        pltpu.SemaphoreType.DMA,
        pltpu.SemaphoreType.DMA,
        pltpu.SemaphoreType.DMA,
        pltpu.SemaphoreType.DMA,
        pltpu.SemaphoreType.DMA,
        pltpu.SemaphoreType.DMA,
        pltpu.SemaphoreType.DMA,
    ],
)
def _emb_lookup(idx_hbm, table_hbm, out_hbm, idx_v,
                buf0, buf1, buf2, buf3,
                gs0, gs1, gs2, gs3, ss0, ss1, ss2, ss3):
    wid = lax.axis_index("s") * NUM_CORES + lax.axis_index("c")
    base = wid * BPW
    pltpu.sync_copy(idx_hbm.at[pl.ds(base, BPW)], idx_v)

    bufs = (buf0, buf1, buf2, buf3)
    gss = (gs0, gs1, gs2, gs3)
    sss = (ss0, ss1, ss2, ss3)

    def start_gather(g, b):
        pltpu.async_copy(table_hbm.at[idx_v.at[pl.ds(g * C, C)]], bufs[b], gss[b])

    def wait_gather(b):
        pltpu.make_async_copy(
            table_hbm.at[idx_v.at[pl.ds(0, C)]], bufs[b], gss[b]
        ).wait()

    def start_scatter(g, b):
        pltpu.async_copy(bufs[b], out_hbm.at[pl.ds(base + g * C, C)], sss[b])

    def wait_scatter(b):
        pltpu.make_async_copy(bufs[b], out_hbm.at[pl.ds(base, C)], sss[b]).wait()

    def scale(b):
        def row_body(r, rc):
            for j in range(D_MODEL // 16):
                sl = pl.ds(j * 16, 16)
                bufs[b][r, sl] = bufs[b][r, sl] * SCALE
            return rc

        lax.fori_loop(0, C, row_body, 0, unroll=False)

    start_gather(0, 0)
    start_gather(1, 1)

    def ring_turn(t, carry):
        for b in range(NBUF):
            g = NBUF * t + b
            wait_gather(b)
            scale(b)
            start_scatter(g, b)
            bn = (b + 2) % NBUF

            @pl.when(g >= 2)
            def _drain():
                wait_scatter(bn)

            @pl.when(g < NCHUNK - 2)
            def _prefetch():
                start_gather(g + 2, bn)

        return carry

    lax.fori_loop(0, T, ring_turn, 0, unroll=False)
    wait_scatter((NCHUNK - 2) % NBUF)
    wait_scatter((NCHUNK - 1) % NBUF)


def kernel(x, table):
    flat_idx = x.reshape(-1)
    out = _emb_lookup(flat_idx, table)
    return out.reshape(x.shape + (D_MODEL,))


# speedup vs baseline: 1.8197x; 1.0076x over previous
import functools
import math

import jax
import jax.numpy as jnp
from jax import lax
from jax.experimental import pallas as pl
from jax.experimental.pallas import tpu as pltpu
from jax.experimental.pallas import tpu_sc as plsc

D_MODEL = 512
SCALE = float(math.sqrt(D_MODEL))

NUM_CORES = 2
NUM_SUBCORES = 16
NW = NUM_CORES * NUM_SUBCORES

B_TOTAL = 1024 * 200
BPW = B_TOTAL // NW
C = 40
NCHUNK = BPW // C
NBUF = 5
PF = 3
T = NCHUNK // NBUF

_mesh = plsc.VectorSubcoreMesh(core_axis_name="c", subcore_axis_name="s")


@functools.partial(
    pl.kernel,
    mesh=_mesh,
    out_type=jax.ShapeDtypeStruct((B_TOTAL, D_MODEL), jnp.float32),
    scratch_types=(
        [pltpu.VMEM((BPW,), jnp.int32)]
        + [pltpu.VMEM((C, D_MODEL), jnp.float32)] * NBUF
        + [pltpu.SemaphoreType.DMA] * (2 * NBUF)
    ),
)
def _emb_lookup(idx_hbm, table_hbm, out_hbm, idx_v, *bufs_and_sems):
    bufs = bufs_and_sems[:NBUF]
    gss = bufs_and_sems[NBUF:2 * NBUF]
    sss = bufs_and_sems[2 * NBUF:]

    wid = lax.axis_index("s") * NUM_CORES + lax.axis_index("c")
    base = wid * BPW
    pltpu.sync_copy(idx_hbm.at[pl.ds(base, BPW)], idx_v)

    def start_gather(g, b):
        pltpu.async_copy(table_hbm.at[idx_v.at[pl.ds(g * C, C)]], bufs[b], gss[b])

    def wait_gather(b):
        pltpu.make_async_copy(
            table_hbm.at[idx_v.at[pl.ds(0, C)]], bufs[b], gss[b]
        ).wait()

    def start_scatter(g, b):
        pltpu.async_copy(bufs[b], out_hbm.at[pl.ds(base + g * C, C)], sss[b])

    def wait_scatter(b):
        pltpu.make_async_copy(bufs[b], out_hbm.at[pl.ds(base, C)], sss[b]).wait()

    def scale(b):
        def row_body(r, rc):
            for j in range(D_MODEL // 16):
                sl = pl.ds(j * 16, 16)
                bufs[b][r, sl] = bufs[b][r, sl] * SCALE
            return rc

        lax.fori_loop(0, C, row_body, 0, unroll=False)

    for g0 in range(PF):
        start_gather(g0, g0)

    def ring_turn(t, carry):
        for b in range(NBUF):
            g = NBUF * t + b
            wait_gather(b)
            scale(b)
            start_scatter(g, b)
            bn = (b + PF) % NBUF

            @pl.when(g >= NBUF - PF)
            def _drain():
                wait_scatter(bn)

            @pl.when(g < NCHUNK - PF)
            def _prefetch():
                start_gather(g + PF, bn)

        return carry

    lax.fori_loop(0, T, ring_turn, 0, unroll=False)
    for g0 in range(NCHUNK - (NBUF - PF), NCHUNK):
        wait_scatter(g0 % NBUF)


def kernel(x, table):
    flat_idx = x.reshape(-1)
    out = _emb_lookup(flat_idx, table)
    return out.reshape(x.shape + (D_MODEL,))
